# Initial kernel scaffold; baseline (speedup 1.0000x reference)
import jax
import jax.numpy as jnp
from jax.experimental import pallas as pl


def kernel(pred_logits, pred_boxes, orig_target_sizes):
    raise NotImplementedError("write your pallas kernel here")


# speedup vs baseline: 8.4732x; 8.4732x over previous
import functools

import jax
import jax.numpy as jnp
from jax import lax
from jax.experimental import pallas as pl
from jax.experimental.pallas import tpu as pltpu
from jax.experimental.pallas import tpu_sc as plsc

B = 32
N = 5000
C = 80
NQ = 300
ROW = N * C
W = 8000
NWIN = ROW // W
CHUNK = 320
CPW = W // CHUNK
NCHUNK = ROW // CHUNK
HB = 4096
CAP1 = 1024
CAPF = 384
DELTA = 16384
INTMIN = -(2 ** 31)
PAD_IDX = 2 ** 24
PAD_VAL = -1e30


def _key_of(bits):
    return jnp.bitwise_xor(
        bits,
        jnp.bitwise_and(
            lax.shift_right_arithmetic(bits, 31), jnp.int32(0x7FFFFFFF)
        ),
    )


def _stream_windows(logits_hbm, row, win0, win1, sem0, sem1, chunk_fn, carry0):
    base = row * ROW

    def src(w):
        return logits_hbm.at[pl.ds(base + w * W, W)]

    pltpu.async_copy(src(0), win0, sem0)
    pltpu.async_copy(src(1), win1, sem1)

    def wbody(wp, carry):
        for half, (win, sem) in enumerate(((win0, sem0), (win1, sem1))):
            w = 2 * wp + half
            pltpu.make_async_copy(src(w), win, sem).wait()

            def cbody(c, cc, _w=w, _win=win):
                return chunk_fn(_w * CPW + c, _win, c, cc)

            carry = lax.fori_loop(0, CPW, cbody, carry)

            @pl.when(wp < NWIN // 2 - 1)
            def _(w=w, win=win, sem=sem):
                pltpu.async_copy(src(w + 2), win, sem)

        return carry

    return lax.fori_loop(0, NWIN // 2, wbody, carry0)


def _suffix_select(hist, totals, csum, need):
    iota = lax.iota(jnp.int32, 16)

    def tbody(i, _):
        bvec = i * 16 + iota
        tv = jnp.zeros((16,), jnp.int32)
        for l in range(16):
            tv = tv + plsc.load_gather(hist, [bvec, jnp.full((16,), l, jnp.int32)])
        totals[pl.ds(i * 16, 16)] = tv
        csum[pl.ds(i * 16, 16)] = jnp.full((16,), jnp.sum(tv), jnp.int32)
        return 0

    lax.fori_loop(0, HB // 16, tbody, 0)

    def sbody(i, carry):
        acc, istar, above = carry
        ii = (HB // 16 - 1) - i
        cs = csum[pl.ds(ii * 16, 16)][0]
        newacc = acc + cs
        hit = jnp.logical_and(istar < 0, newacc >= need)
        istar = jnp.where(hit, ii, istar)
        above = jnp.where(hit, acc, above)
        return (newacc, istar, above)

    _, istar, above = lax.fori_loop(
        0, HB // 16, sbody,
        (jnp.int32(0), jnp.int32(-1), jnp.int32(0)))
    istar = jnp.maximum(istar, 0)

    tv = totals[pl.ds(istar * 16, 16)]
    suff = lax.rev(jnp.cumsum(lax.rev(tv, (0,))), (0,)) + above
    mask = suff >= need
    ntrue = jnp.max(plsc.all_reduce_population_count(mask))
    bloc = jnp.maximum(ntrue - 1, 0)
    return istar * 16 + bloc


def _sc_body(logits_hbm, boxes_hbm, out_v, out_i, out_b,
             win0, win1, hist, cmax, totals, csum,
             cand_v, cand_i, fin_v, fin_i, bidxv, fin_g, fb4,
             sem0, sem1, semg):
    row = lax.axis_index("s") * 2 + lax.axis_index("c")
    iota = lax.iota(jnp.int32, 16)
    ones = jnp.ones((16,), jnp.int32)

    def zb(i, _):
        hist[i] = jnp.zeros((16,), jnp.int32)
        return 0
    lax.fori_loop(0, HB, zb, 0)

    def p1_chunk(g, win, c, carry):
        runmax = jnp.full((16,), -jnp.inf, jnp.float32)
        for j in range(CHUNK // 16):
            v = win[pl.ds(c * CHUNK + j * 16, 16)]
            key = _key_of(lax.bitcast_convert_type(v, jnp.int32))
            b = lax.shift_right_arithmetic(key, 20) + 2048
            plsc.addupdate_scatter(hist, [b, iota], ones)
            runmax = jnp.maximum(runmax, v)
        cmax[g] = runmax
        return carry

    _stream_windows(logits_hbm, row, win0, win1, sem0, sem1, p1_chunk,
                    jnp.int32(0))

    b1 = _suffix_select(hist, totals, csum, NQ)
    tlo = lax.shift_left(b1 - 2048, 20)
    tlo_d = jnp.where(tlo < INTMIN + DELTA + 1, jnp.int32(INTMIN + 1),
                      tlo - DELTA)
    tvec = jnp.full((16,), tlo_d, jnp.int32)
    tf = jnp.max(lax.bitcast_convert_type(_key_of(tvec), jnp.float32))

    def ib(q, _):
        cand_v[pl.ds(q * 16, 16)] = jnp.full((16,), PAD_VAL, jnp.float32)
        cand_i[pl.ds(q * 16, 16)] = jnp.full((16,), PAD_IDX, jnp.int32)
        return 0
    lax.fori_loop(0, CAP1 // 16, ib, 0)

    def p2_chunk(g, win, c, off):
        def hot(off):
            for j in range(CHUNK // 16):
                v = win[pl.ds(c * CHUNK + j * 16, 16)]
                key = _key_of(lax.bitcast_convert_type(v, jnp.int32))
                m = key >= tlo_d

                def put(off, v=v, m=m, j=j):
                    pos = off + jnp.cumsum(m.astype(jnp.int32)) - 1
                    pos = jnp.minimum(pos, CAP1 - 1)
                    plsc.store_scatter(cand_v, [pos], v, mask=m)
                    gidx = g * CHUNK + j * 16 + iota
                    plsc.store_scatter(cand_i, [pos], gidx, mask=m)
                    return off + plsc.all_reduce_population_count(m)

                off = lax.cond(jnp.any(m), put, lambda off: off, off)
            return off

        return lax.cond(jnp.any(cmax[g] >= tf), hot, lambda off: off, off)

    _stream_windows(logits_hbm, row, win0, win1, sem0, sem1, p2_chunk,
                    jnp.zeros((16,), jnp.int32))

    def zb2(i, _):
        hist[i] = jnp.zeros((16,), jnp.int32)
        return 0
    lax.fori_loop(0, HB, zb2, 0)

    clamp_hi = lax.shift_left(jnp.minimum(b1, 4094) + 1 - 2048, 20)

    def rbody(q, _):
        v = cand_v[pl.ds(q * 16, 16)]
        key = _key_of(lax.bitcast_convert_type(v, jnp.int32))
        valid = key >= tlo_d
        keyc = jnp.clip(key, tlo_d, clamp_hi)
        b2v = lax.shift_right_arithmetic(keyc - tlo_d, 9)
        plsc.addupdate_scatter(hist, [b2v, iota], ones, mask=valid)
        return 0
    lax.fori_loop(0, CAP1 // 16, rbody, 0)

    b2 = _suffix_select(hist, totals, csum, NQ)
    t2 = tlo_d + lax.shift_left(b2, 9)
    t2_d = jnp.where(t2 < INTMIN + DELTA + 1, jnp.int32(INTMIN + 1),
                     t2 - DELTA)

    def fb(q, _):
        fin_v[pl.ds(q * 16, 16)] = jnp.full((16,), PAD_VAL, jnp.float32)
        fin_i[pl.ds(q * 16, 16)] = jnp.full((16,), PAD_IDX, jnp.int32)
        return 0
    lax.fori_loop(0, CAPF // 16, fb, 0)

    def cbody(q, off):
        v = cand_v[pl.ds(q * 16, 16)]
        idx = cand_i[pl.ds(q * 16, 16)]
        key = _key_of(lax.bitcast_convert_type(v, jnp.int32))
        m = key >= t2_d

        def put(off):
            pos = off + jnp.cumsum(m.astype(jnp.int32)) - 1
            pos = jnp.minimum(pos, CAPF - 1)
            plsc.store_scatter(fin_v, [pos], v, mask=m)
            plsc.store_scatter(fin_i, [pos], idx, mask=m)
            return off + plsc.all_reduce_population_count(m)

        return lax.cond(jnp.any(m), put, lambda off: off, off)

    lax.fori_loop(0, CAP1 // 16, cbody, jnp.zeros((16,), jnp.int32))

    def xbody(q, _):
        iv = fin_i[pl.ds(q * 16, 16)]
        g = row * N + jnp.minimum(iv // C, N - 1)
        bidxv[q // 8, pl.ds((q % 8) * 16, 16)] = g // 4
        return 0
    lax.fori_loop(0, CAPF // 16, xbody, 0)

    copies = []
    for k in range(CAPF // 128):
        copies.append(pltpu.async_copy(
            boxes_hbm.at[bidxv.at[k]],
            fin_g.at[pl.ds(k * 128, 128)], semg))
    for cp in copies:
        cp.wait()

    def ub(q, _):
        iv = fin_i[pl.ds(q * 16, 16)]
        g = row * N + jnp.minimum(iv // C, N - 1)
        rows = q * 16 + iota
        sub = (g % 4) * 4
        for comp in range(4):
            fb4[comp, pl.ds(q * 16, 16)] = plsc.load_gather(
                fin_g, [rows, sub + comp])
        return 0
    lax.fori_loop(0, CAPF // 16, ub, 0)

    pltpu.sync_copy(fin_v, out_v.at[row])
    pltpu.sync_copy(fin_i, out_i.at[row])
    pltpu.sync_copy(fb4, out_b.at[row])


@jax.jit
def _sc_select(logits_flat, boxes_tbl):
    return pl.kernel(
        _sc_body,
        out_type=[
            jax.ShapeDtypeStruct((B, CAPF), jnp.float32),
            jax.ShapeDtypeStruct((B, CAPF), jnp.int32),
            jax.ShapeDtypeStruct((B, 4, CAPF), jnp.float32),
        ],
        mesh=plsc.VectorSubcoreMesh(core_axis_name="c", subcore_axis_name="s"),
        compiler_params=pltpu.CompilerParams(
            needs_layout_passes=False, use_tc_tiling_on_sc=False),
        scratch_types=[
            pltpu.VMEM((W,), jnp.float32),
            pltpu.VMEM((W,), jnp.float32),
            pltpu.VMEM((HB, 16), jnp.int32),
            pltpu.VMEM((NCHUNK, 16), jnp.float32),
            pltpu.VMEM((HB,), jnp.int32),
            pltpu.VMEM((HB,), jnp.int32),
            pltpu.VMEM((CAP1,), jnp.float32),
            pltpu.VMEM((CAP1,), jnp.int32),
            pltpu.VMEM((CAPF,), jnp.float32),
            pltpu.VMEM((CAPF,), jnp.int32),
            pltpu.VMEM((CAPF // 128, 128), jnp.int32),
            pltpu.VMEM((CAPF, 16), jnp.float32),
            pltpu.VMEM((4, CAPF), jnp.float32),
            pltpu.SemaphoreType.DMA,
            pltpu.SemaphoreType.DMA,
            pltpu.SemaphoreType.DMA,
        ],
    )(logits_flat, boxes_tbl)


def _tc_body(score_ref, idx_ref, boxes_ref, size_ref,
             lab_ref, box_ref, sc_ref):
    s = score_ref[0, 0]
    idx = idx_ref[0, 0]
    idxf = idx.astype(jnp.float32)

    srow = s[None, :]
    scol = s[:, None]
    irow = idxf[None, :]
    icol = idxf[:, None]
    better = jnp.logical_or(
        srow > scol, jnp.logical_and(srow == scol, irow < icol))
    ranks = jnp.sum(better.astype(jnp.int32), axis=1)

    prow = lax.broadcasted_iota(jnp.int32, (CAPF, CAPF), 0)
    onehot = (prow == ranks[None, :]).astype(jnp.float32)

    def pick(col):
        return jnp.sum(onehot * col[None, :], axis=1)

    bx = boxes_ref[0]
    top_s = pick(s)[:NQ]
    top_i = pick(idxf)[:NQ].astype(jnp.int32)
    labels = top_i % C

    cx = pick(bx[:, 0])[:NQ]
    cy = pick(bx[:, 1])[:NQ]
    w = pick(bx[:, 2])[:NQ]
    h = pick(bx[:, 3])[:NQ]
    xyxy = jnp.stack(
        [cx - 0.5 * w, cy - 0.5 * h, cx + 0.5 * w, cy + 0.5 * h], axis=-1)
    w0 = size_ref[0, 0, 0]
    h0 = size_ref[0, 0, 1]
    scale = jnp.stack([w0, h0, w0, h0])[None, :]
    xyxy = xyxy * scale

    lab_ref[0, 0] = labels
    box_ref[0] = xyxy
    sc_ref[0, 0] = top_s


@jax.jit
def _tc_finalize(scores, idxs, boxes, sizes):
    labels, xyxy, top_s = pl.pallas_call(
        _tc_body,
        grid=(B,),
        in_specs=[
            pl.BlockSpec((1, 1, CAPF), lambda r: (r, 0, 0)),
            pl.BlockSpec((1, 1, CAPF), lambda r: (r, 0, 0)),
            pl.BlockSpec((1, CAPF, 4), lambda r: (r, 0, 0)),
            pl.BlockSpec((1, 1, 2), lambda r: (r, 0, 0)),
        ],
        out_specs=[
            pl.BlockSpec((1, 1, NQ), lambda r: (r, 0, 0)),
            pl.BlockSpec((1, NQ, 4), lambda r: (r, 0, 0)),
            pl.BlockSpec((1, 1, NQ), lambda r: (r, 0, 0)),
        ],
        out_shape=[
            jax.ShapeDtypeStruct((B, 1, NQ), jnp.int32),
            jax.ShapeDtypeStruct((B, NQ, 4), jnp.float32),
            jax.ShapeDtypeStruct((B, 1, NQ), jnp.float32),
        ],
    )(scores.reshape(B, 1, CAPF), idxs.reshape(B, 1, CAPF), boxes,
      sizes.reshape(B, 1, 2))
    return labels.reshape(B, NQ), xyxy, top_s.reshape(B, NQ)


def kernel(pred_logits, pred_boxes, orig_target_sizes):
    logits_flat = pred_logits.reshape(B * ROW)
    boxes_tbl = pred_boxes.reshape(B * N // 4, 16)
    cand_v, cand_i, cand_b = _sc_select(logits_flat, boxes_tbl)
    cand_b = cand_b.reshape(B, 4, CAPF).transpose(0, 2, 1)
    cand_s = jax.nn.sigmoid(cand_v)
    labels, boxes, scores = _tc_finalize(cand_s, cand_i, cand_b,
                                         orig_target_sizes)
    return (labels, boxes, scores)


# speedup vs baseline: 10.8895x; 1.2852x over previous
import functools

import jax
import jax.numpy as jnp
from jax import lax
from jax.experimental import pallas as pl
from jax.experimental.pallas import tpu as pltpu
from jax.experimental.pallas import tpu_sc as plsc

B = 32
N = 5000
C = 80
NQ = 300
ROW = N * C
W = 8000
NWIN = ROW // W
CHUNK = 320
CPW = W // CHUNK
NCHUNK = ROW // CHUNK
HB = 4096
CAP1 = 1024
CAPF = 384
DELTA = 16384
INTMIN = -(2 ** 31)
PAD_IDX = 2 ** 24
PAD_VAL = -1e30


def _key_of(bits):
    return jnp.bitwise_xor(
        bits,
        jnp.bitwise_and(
            lax.shift_right_arithmetic(bits, 31), jnp.int32(0x7FFFFFFF)
        ),
    )


def _stream_windows(logits_hbm, row, win0, win1, sem0, sem1, chunk_fn, carry0):
    base = row * ROW

    def src(w):
        return logits_hbm.at[pl.ds(base + w * W, W)]

    pltpu.async_copy(src(0), win0, sem0)
    pltpu.async_copy(src(1), win1, sem1)

    def wbody(wp, carry):
        for half, (win, sem) in enumerate(((win0, sem0), (win1, sem1))):
            w = 2 * wp + half
            pltpu.make_async_copy(src(w), win, sem).wait()

            def cbody(c, cc, _w=w, _win=win):
                return chunk_fn(_w * CPW + c, _win, c, cc)

            carry = lax.fori_loop(0, CPW, cbody, carry)

            @pl.when(wp < NWIN // 2 - 1)
            def _(w=w, win=win, sem=sem):
                pltpu.async_copy(src(w + 2), win, sem)

        return carry

    return lax.fori_loop(0, NWIN // 2, wbody, carry0)


def _suffix_select(hist, totals, csum, need):
    iota = lax.iota(jnp.int32, 16)

    def tbody(i, _):
        bvec = i * 16 + iota
        tv = jnp.zeros((16,), jnp.int32)
        for l in range(16):
            tv = tv + plsc.load_gather(hist, [bvec, jnp.full((16,), l, jnp.int32)])
        totals[pl.ds(i * 16, 16)] = tv
        csum[pl.ds(i * 16, 16)] = jnp.full((16,), jnp.sum(tv), jnp.int32)
        return 0

    lax.fori_loop(0, HB // 16, tbody, 0)

    def sbody(i, carry):
        acc, istar, above = carry
        ii = (HB // 16 - 1) - i
        cs = csum[pl.ds(ii * 16, 16)][0]
        newacc = acc + cs
        hit = jnp.logical_and(istar < 0, newacc >= need)
        istar = jnp.where(hit, ii, istar)
        above = jnp.where(hit, acc, above)
        return (newacc, istar, above)

    _, istar, above = lax.fori_loop(
        0, HB // 16, sbody,
        (jnp.int32(0), jnp.int32(-1), jnp.int32(0)))
    istar = jnp.maximum(istar, 0)

    tv = totals[pl.ds(istar * 16, 16)]
    suff = lax.rev(jnp.cumsum(lax.rev(tv, (0,))), (0,)) + above
    mask = suff >= need
    ntrue = jnp.max(plsc.all_reduce_population_count(mask))
    bloc = jnp.maximum(ntrue - 1, 0)
    return istar * 16 + bloc


def _sc_body(logits_hbm, boxes_hbm, out_v, out_i, out_b,
             win0, win1, hist, cmax, totals, csum,
             cand_v, cand_i, fin_v, fin_i, bidxv, fin_g, fb4,
             sem0, sem1, semg):
    row = lax.axis_index("s") * 2 + lax.axis_index("c")
    iota = lax.iota(jnp.int32, 16)
    ones = jnp.ones((16,), jnp.int32)

    zvec = jnp.zeros((16,), jnp.int32)

    def zb(i, _):
        for k in range(8):
            hist[i * 8 + k] = zvec
        return 0
    lax.fori_loop(0, HB // 8, zb, 0)

    def p1_chunk(g, win, c, carry):
        runmax = jnp.full((16,), -jnp.inf, jnp.float32)
        for j in range(CHUNK // 16):
            v = win[pl.ds(c * CHUNK + j * 16, 16)]
            key = _key_of(lax.bitcast_convert_type(v, jnp.int32))
            b = lax.shift_right_arithmetic(key, 20) + 2048
            plsc.addupdate_scatter(hist, [b, iota], ones)
            runmax = jnp.maximum(runmax, v)
        cmax[g] = runmax
        return carry

    _stream_windows(logits_hbm, row, win0, win1, sem0, sem1, p1_chunk,
                    jnp.int32(0))

    b1 = _suffix_select(hist, totals, csum, NQ)
    tlo = lax.shift_left(b1 - 2048, 20)
    tlo_d = jnp.where(tlo < INTMIN + DELTA + 1, jnp.int32(INTMIN + 1),
                      tlo - DELTA)
    tvec = jnp.full((16,), tlo_d, jnp.int32)
    tf = jnp.max(lax.bitcast_convert_type(_key_of(tvec), jnp.float32))

    def ib(q, _):
        cand_v[pl.ds(q * 16, 16)] = jnp.full((16,), PAD_VAL, jnp.float32)
        cand_i[pl.ds(q * 16, 16)] = jnp.full((16,), PAD_IDX, jnp.int32)
        return 0
    lax.fori_loop(0, CAP1 // 16, ib, 0)

    def p2_chunk(g, win, c, off):
        def hot(off):
            for jg in range(CHUNK // 64):
                grp = []
                for j4 in range(4):
                    j = jg * 4 + j4
                    v = win[pl.ds(c * CHUNK + j * 16, 16)]
                    key = _key_of(lax.bitcast_convert_type(v, jnp.int32))
                    grp.append((j, v, key >= tlo_d))
                mor = grp[0][2] | grp[1][2] | grp[2][2] | grp[3][2]

                def dogrp(off, grp=grp):
                    for j, v, m in grp:
                        def put(off, v=v, m=m, j=j):
                            pos = off + jnp.cumsum(m.astype(jnp.int32)) - 1
                            pos = jnp.minimum(pos, CAP1 - 1)
                            plsc.store_scatter(cand_v, [pos], v, mask=m)
                            gidx = g * CHUNK + j * 16 + iota
                            plsc.store_scatter(cand_i, [pos], gidx, mask=m)
                            return off + plsc.all_reduce_population_count(m)

                        off = lax.cond(jnp.any(m), put, lambda off: off, off)
                    return off

                off = lax.cond(jnp.any(mor), dogrp, lambda off: off, off)
            return off

        return lax.cond(jnp.any(cmax[g] >= tf), hot, lambda off: off, off)

    _stream_windows(logits_hbm, row, win0, win1, sem0, sem1, p2_chunk,
                    jnp.zeros((16,), jnp.int32))

    def zb2(i, _):
        for k in range(8):
            hist[i * 8 + k] = zvec
        return 0
    lax.fori_loop(0, HB // 8, zb2, 0)

    clamp_hi = lax.shift_left(jnp.minimum(b1, 4094) + 1 - 2048, 20)

    def rbody(q, _):
        v = cand_v[pl.ds(q * 16, 16)]
        key = _key_of(lax.bitcast_convert_type(v, jnp.int32))
        valid = key >= tlo_d
        keyc = jnp.clip(key, tlo_d, clamp_hi)
        b2v = lax.shift_right_arithmetic(keyc - tlo_d, 9)
        plsc.addupdate_scatter(hist, [b2v, iota], ones, mask=valid)
        return 0
    lax.fori_loop(0, CAP1 // 16, rbody, 0)

    b2 = _suffix_select(hist, totals, csum, NQ)
    t2 = tlo_d + lax.shift_left(b2, 9)
    t2_d = jnp.where(t2 < INTMIN + DELTA + 1, jnp.int32(INTMIN + 1),
                     t2 - DELTA)

    def fb(q, _):
        fin_v[pl.ds(q * 16, 16)] = jnp.full((16,), PAD_VAL, jnp.float32)
        fin_i[pl.ds(q * 16, 16)] = jnp.full((16,), PAD_IDX, jnp.int32)
        return 0
    lax.fori_loop(0, CAPF // 16, fb, 0)

    def cbody(q, off):
        v = cand_v[pl.ds(q * 16, 16)]
        idx = cand_i[pl.ds(q * 16, 16)]
        key = _key_of(lax.bitcast_convert_type(v, jnp.int32))
        m = key >= t2_d

        def put(off):
            pos = off + jnp.cumsum(m.astype(jnp.int32)) - 1
            pos = jnp.minimum(pos, CAPF - 1)
            plsc.store_scatter(fin_v, [pos], v, mask=m)
            plsc.store_scatter(fin_i, [pos], idx, mask=m)
            return off + plsc.all_reduce_population_count(m)

        return lax.cond(jnp.any(m), put, lambda off: off, off)

    lax.fori_loop(0, CAP1 // 16, cbody, jnp.zeros((16,), jnp.int32))

    def xbody(q, _):
        iv = fin_i[pl.ds(q * 16, 16)]
        g = row * N + jnp.minimum(iv // C, N - 1)
        bidxv[q // 8, pl.ds((q % 8) * 16, 16)] = g // 4
        return 0
    lax.fori_loop(0, CAPF // 16, xbody, 0)

    copies = []
    for k in range(CAPF // 128):
        copies.append(pltpu.async_copy(
            boxes_hbm.at[bidxv.at[k]],
            fin_g.at[pl.ds(k * 128, 128)], semg))
    for cp in copies:
        cp.wait()

    def ub(q, _):
        iv = fin_i[pl.ds(q * 16, 16)]
        g = row * N + jnp.minimum(iv // C, N - 1)
        rows = q * 16 + iota
        sub = (g % 4) * 4
        for comp in range(4):
            fb4[comp, pl.ds(q * 16, 16)] = plsc.load_gather(
                fin_g, [rows, sub + comp])
        return 0
    lax.fori_loop(0, CAPF // 16, ub, 0)

    pltpu.sync_copy(fin_v, out_v.at[row])
    pltpu.sync_copy(fin_i, out_i.at[row])
    pltpu.sync_copy(fb4, out_b.at[row])


@jax.jit
def _sc_select(logits_flat, boxes_tbl):
    return pl.kernel(
        _sc_body,
        out_type=[
            jax.ShapeDtypeStruct((B, CAPF), jnp.float32),
            jax.ShapeDtypeStruct((B, CAPF), jnp.int32),
            jax.ShapeDtypeStruct((B, 4, CAPF), jnp.float32),
        ],
        mesh=plsc.VectorSubcoreMesh(core_axis_name="c", subcore_axis_name="s"),
        compiler_params=pltpu.CompilerParams(
            needs_layout_passes=False, use_tc_tiling_on_sc=False),
        scratch_types=[
            pltpu.VMEM((W,), jnp.float32),
            pltpu.VMEM((W,), jnp.float32),
            pltpu.VMEM((HB, 16), jnp.int32),
            pltpu.VMEM((NCHUNK, 16), jnp.float32),
            pltpu.VMEM((HB,), jnp.int32),
            pltpu.VMEM((HB,), jnp.int32),
            pltpu.VMEM((CAP1,), jnp.float32),
            pltpu.VMEM((CAP1,), jnp.int32),
            pltpu.VMEM((CAPF,), jnp.float32),
            pltpu.VMEM((CAPF,), jnp.int32),
            pltpu.VMEM((CAPF // 128, 128), jnp.int32),
            pltpu.VMEM((CAPF, 16), jnp.float32),
            pltpu.VMEM((4, CAPF), jnp.float32),
            pltpu.SemaphoreType.DMA,
            pltpu.SemaphoreType.DMA,
            pltpu.SemaphoreType.DMA,
        ],
    )(logits_flat, boxes_tbl)


def _tc_body(score_ref, idx_ref, boxes_ref, size_ref,
             lab_ref, box_ref, sc_ref):
    s = score_ref[0, 0]
    idx = idx_ref[0, 0]
    idxf = idx.astype(jnp.float32)

    srow = s[None, :]
    scol = s[:, None]
    irow = idxf[None, :]
    icol = idxf[:, None]
    better = jnp.logical_or(
        srow > scol, jnp.logical_and(srow == scol, irow < icol))
    ranks = jnp.sum(better.astype(jnp.int32), axis=1)

    prow = lax.broadcasted_iota(jnp.int32, (CAPF, CAPF), 0)
    onehot = (prow == ranks[None, :]).astype(jnp.float32)

    def pick(col):
        return jnp.sum(onehot * col[None, :], axis=1)

    bx = boxes_ref[0]
    top_s = pick(s)[:NQ]
    top_i = pick(idxf)[:NQ].astype(jnp.int32)
    labels = top_i % C

    cx = pick(bx[:, 0])[:NQ]
    cy = pick(bx[:, 1])[:NQ]
    w = pick(bx[:, 2])[:NQ]
    h = pick(bx[:, 3])[:NQ]
    xyxy = jnp.stack(
        [cx - 0.5 * w, cy - 0.5 * h, cx + 0.5 * w, cy + 0.5 * h], axis=-1)
    w0 = size_ref[0, 0, 0]
    h0 = size_ref[0, 0, 1]
    scale = jnp.stack([w0, h0, w0, h0])[None, :]
    xyxy = xyxy * scale

    lab_ref[0, 0] = labels
    box_ref[0] = xyxy
    sc_ref[0, 0] = top_s


@jax.jit
def _tc_finalize(scores, idxs, boxes, sizes):
    labels, xyxy, top_s = pl.pallas_call(
        _tc_body,
        grid=(B,),
        in_specs=[
            pl.BlockSpec((1, 1, CAPF), lambda r: (r, 0, 0)),
            pl.BlockSpec((1, 1, CAPF), lambda r: (r, 0, 0)),
            pl.BlockSpec((1, CAPF, 4), lambda r: (r, 0, 0)),
            pl.BlockSpec((1, 1, 2), lambda r: (r, 0, 0)),
        ],
        out_specs=[
            pl.BlockSpec((1, 1, NQ), lambda r: (r, 0, 0)),
            pl.BlockSpec((1, NQ, 4), lambda r: (r, 0, 0)),
            pl.BlockSpec((1, 1, NQ), lambda r: (r, 0, 0)),
        ],
        out_shape=[
            jax.ShapeDtypeStruct((B, 1, NQ), jnp.int32),
            jax.ShapeDtypeStruct((B, NQ, 4), jnp.float32),
            jax.ShapeDtypeStruct((B, 1, NQ), jnp.float32),
        ],
    )(scores.reshape(B, 1, CAPF), idxs.reshape(B, 1, CAPF), boxes,
      sizes.reshape(B, 1, 2))
    return labels.reshape(B, NQ), xyxy, top_s.reshape(B, NQ)


def kernel(pred_logits, pred_boxes, orig_target_sizes):
    logits_flat = pred_logits.reshape(B * ROW)
    boxes_tbl = pred_boxes.reshape(B * N // 4, 16)
    cand_v, cand_i, cand_b = _sc_select(logits_flat, boxes_tbl)
    cand_b = cand_b.reshape(B, 4, CAPF).transpose(0, 2, 1)
    cand_s = jax.nn.sigmoid(cand_v)
    labels, boxes, scores = _tc_finalize(cand_s, cand_i, cand_b,
                                         orig_target_sizes)
    return (labels, boxes, scores)


# speedup vs baseline: 10.8946x; 1.0005x over previous
import jax
import jax.numpy as jnp
from jax import lax
from jax.experimental import pallas as pl
from jax.experimental.pallas import tpu as pltpu
from jax.experimental.pallas import tpu_sc as plsc

B = 32
N = 5000
C = 80
NQ = 300
ROW = N * C
W = 8000
NWIN = ROW // W
CHUNK = 320
CPW = W // CHUNK
NCHUNK = ROW // CHUNK
HB = 4096
CAP1 = 1024
CAPF = 384
DELTA = 16384
INTMIN = -(2 ** 31)
PAD_IDX = 2 ** 24
PAD_VAL = -1e30


def _key_of(bits):
    return jnp.bitwise_xor(
        bits,
        jnp.bitwise_and(
            lax.shift_right_arithmetic(bits, 31), jnp.int32(0x7FFFFFFF)
        ),
    )


def _stream_windows(logits_hbm, row, win0, win1, sem0, sem1, chunk_fn, carry0):
    base = row * ROW

    def src(w):
        return logits_hbm.at[pl.ds(base + w * W, W)]

    pltpu.async_copy(src(0), win0, sem0)
    pltpu.async_copy(src(1), win1, sem1)

    def wbody(wp, carry):
        for half, (win, sem) in enumerate(((win0, sem0), (win1, sem1))):
            w = 2 * wp + half
            pltpu.make_async_copy(src(w), win, sem).wait()

            def cbody(c, cc, _w=w, _win=win):
                return chunk_fn(_w * CPW + c, _win, c, cc)

            carry = lax.fori_loop(0, CPW, cbody, carry)

            @pl.when(wp < NWIN // 2 - 1)
            def _(w=w, win=win, sem=sem):
                pltpu.async_copy(src(w + 2), win, sem)

        return carry

    return lax.fori_loop(0, NWIN // 2, wbody, carry0)


def _suffix_select(hist, totals, csum, need):
    iota = lax.iota(jnp.int32, 16)

    def tbody(i, _):
        bvec = i * 16 + iota
        tv = jnp.zeros((16,), jnp.int32)
        for l in range(16):
            tv = tv + plsc.load_gather(hist, [bvec, jnp.full((16,), l, jnp.int32)])
        totals[pl.ds(i * 16, 16)] = tv
        csum[pl.ds(i * 16, 16)] = jnp.full((16,), jnp.sum(tv), jnp.int32)
        return 0

    lax.fori_loop(0, HB // 16, tbody, 0)

    def sbody(i, carry):
        acc, istar, above = carry
        ii = (HB // 16 - 1) - i
        cs = csum[pl.ds(ii * 16, 16)][0]
        newacc = acc + cs
        hit = jnp.logical_and(istar < 0, newacc >= need)
        istar = jnp.where(hit, ii, istar)
        above = jnp.where(hit, acc, above)
        return (newacc, istar, above)

    _, istar, above = lax.fori_loop(
        0, HB // 16, sbody,
        (jnp.int32(0), jnp.int32(-1), jnp.int32(0)))
    istar = jnp.maximum(istar, 0)

    tv = totals[pl.ds(istar * 16, 16)]
    suff = lax.rev(jnp.cumsum(lax.rev(tv, (0,))), (0,)) + above
    mask = suff >= need
    ntrue = jnp.max(plsc.all_reduce_population_count(mask))
    bloc = jnp.maximum(ntrue - 1, 0)
    return istar * 16 + bloc


def _sc_body(logits_hbm, boxes_hbm, out_v, out_i, out_b,
             win0, win1, hist, cmax, totals, csum,
             cand_v, cand_i, fin_v, fin_i, bidxv, fin_g, fb4,
             sem0, sem1, semg):
    row = lax.axis_index("s") * 2 + lax.axis_index("c")
    iota = lax.iota(jnp.int32, 16)
    ones = jnp.ones((16,), jnp.int32)

    zvec = jnp.zeros((16,), jnp.int32)

    def zb(i, _):
        for k in range(8):
            hist[i * 8 + k] = zvec
        return 0
    lax.fori_loop(0, HB // 8, zb, 0)

    def p1_chunk(g, win, c, carry):
        runmax = jnp.full((16,), -jnp.inf, jnp.float32)
        for j in range(CHUNK // 16):
            v = win[pl.ds(c * CHUNK + j * 16, 16)]
            key = _key_of(lax.bitcast_convert_type(v, jnp.int32))
            b = lax.shift_right_arithmetic(key, 20) + 2048
            plsc.addupdate_scatter(hist, [b, iota], ones)
            runmax = jnp.maximum(runmax, v)
        cmax[g] = runmax
        return carry

    _stream_windows(logits_hbm, row, win0, win1, sem0, sem1, p1_chunk,
                    jnp.int32(0))

    b1 = _suffix_select(hist, totals, csum, NQ)
    tlo = lax.shift_left(b1 - 2048, 20)
    tlo_d = jnp.where(tlo < INTMIN + DELTA + 1, jnp.int32(INTMIN + 1),
                      tlo - DELTA)
    tvec = jnp.full((16,), tlo_d, jnp.int32)
    tf = jnp.max(lax.bitcast_convert_type(_key_of(tvec), jnp.float32))

    def ib(q, _):
        cand_v[pl.ds(q * 16, 16)] = jnp.full((16,), PAD_VAL, jnp.float32)
        cand_i[pl.ds(q * 16, 16)] = jnp.full((16,), PAD_IDX, jnp.int32)
        return 0
    lax.fori_loop(0, CAP1 // 16, ib, 0)

    def p2_chunk(g, win, c, off):
        def hot(off):
            for jg in range(CHUNK // 64):
                grp = []
                for j4 in range(4):
                    j = jg * 4 + j4
                    v = win[pl.ds(c * CHUNK + j * 16, 16)]
                    key = _key_of(lax.bitcast_convert_type(v, jnp.int32))
                    grp.append((j, v, key >= tlo_d))
                mor = grp[0][2] | grp[1][2] | grp[2][2] | grp[3][2]

                def dogrp(off, grp=grp):
                    for j, v, m in grp:
                        def put(off, v=v, m=m, j=j):
                            pos = off + jnp.cumsum(m.astype(jnp.int32)) - 1
                            pos = jnp.minimum(pos, CAP1 - 1)
                            plsc.store_scatter(cand_v, [pos], v, mask=m)
                            gidx = g * CHUNK + j * 16 + iota
                            plsc.store_scatter(cand_i, [pos], gidx, mask=m)
                            return off + plsc.all_reduce_population_count(m)

                        off = lax.cond(jnp.any(m), put, lambda off: off, off)
                    return off

                off = lax.cond(jnp.any(mor), dogrp, lambda off: off, off)
            return off

        return lax.cond(jnp.any(cmax[g] >= tf), hot, lambda off: off, off)

    _stream_windows(logits_hbm, row, win0, win1, sem0, sem1, p2_chunk,
                    jnp.zeros((16,), jnp.int32))

    def zb2(i, _):
        for k in range(8):
            hist[i * 8 + k] = zvec
        return 0
    lax.fori_loop(0, HB // 8, zb2, 0)

    clamp_hi = lax.shift_left(jnp.minimum(b1, 4094) + 1 - 2048, 20)

    def rbody(q, _):
        v = cand_v[pl.ds(q * 16, 16)]
        key = _key_of(lax.bitcast_convert_type(v, jnp.int32))
        valid = key >= tlo_d
        keyc = jnp.clip(key, tlo_d, clamp_hi)
        b2v = lax.shift_right_arithmetic(keyc - tlo_d, 9)
        plsc.addupdate_scatter(hist, [b2v, iota], ones, mask=valid)
        return 0
    lax.fori_loop(0, CAP1 // 16, rbody, 0)

    b2 = _suffix_select(hist, totals, csum, NQ)
    t2 = tlo_d + lax.shift_left(b2, 9)
    t2_d = jnp.where(t2 < INTMIN + DELTA + 1, jnp.int32(INTMIN + 1),
                     t2 - DELTA)

    def fb(q, _):
        fin_v[pl.ds(q * 16, 16)] = jnp.full((16,), PAD_VAL, jnp.float32)
        fin_i[pl.ds(q * 16, 16)] = jnp.full((16,), PAD_IDX, jnp.int32)
        return 0
    lax.fori_loop(0, CAPF // 16, fb, 0)

    def cbody(q, off):
        v = cand_v[pl.ds(q * 16, 16)]
        idx = cand_i[pl.ds(q * 16, 16)]
        key = _key_of(lax.bitcast_convert_type(v, jnp.int32))
        m = key >= t2_d

        def put(off):
            pos = off + jnp.cumsum(m.astype(jnp.int32)) - 1
            pos = jnp.minimum(pos, CAPF - 1)
            plsc.store_scatter(fin_v, [pos], v, mask=m)
            plsc.store_scatter(fin_i, [pos], idx, mask=m)
            return off + plsc.all_reduce_population_count(m)

        return lax.cond(jnp.any(m), put, lambda off: off, off)

    lax.fori_loop(0, CAP1 // 16, cbody, jnp.zeros((16,), jnp.int32))

    def xbody(q, _):
        iv = fin_i[pl.ds(q * 16, 16)]
        g = row * N + jnp.minimum(iv // C, N - 1)
        bidxv[q // 8, pl.ds((q % 8) * 16, 16)] = g // 4
        return 0
    lax.fori_loop(0, CAPF // 16, xbody, 0)

    copies = []
    for k in range(CAPF // 128):
        copies.append(pltpu.async_copy(
            boxes_hbm.at[bidxv.at[k]],
            fin_g.at[pl.ds(k * 128, 128)], semg))
    for cp in copies:
        cp.wait()

    def ub(q, _):
        iv = fin_i[pl.ds(q * 16, 16)]
        g = row * N + jnp.minimum(iv // C, N - 1)
        rows = q * 16 + iota
        sub = (g % 4) * 4
        for comp in range(4):
            fb4[comp, pl.ds(q * 16, 16)] = plsc.load_gather(
                fin_g, [rows, sub + comp])
        return 0
    lax.fori_loop(0, CAPF // 16, ub, 0)

    pltpu.sync_copy(fin_v, out_v.at[row])
    pltpu.sync_copy(fin_i, out_i.at[row])
    pltpu.sync_copy(fb4, out_b.at[row])


@jax.jit
def _sc_select(logits_flat, boxes_tbl):
    return pl.kernel(
        _sc_body,
        out_type=[
            jax.ShapeDtypeStruct((B, CAPF), jnp.float32),
            jax.ShapeDtypeStruct((B, CAPF), jnp.int32),
            jax.ShapeDtypeStruct((B, 4, CAPF), jnp.float32),
        ],
        mesh=plsc.VectorSubcoreMesh(core_axis_name="c", subcore_axis_name="s"),
        compiler_params=pltpu.CompilerParams(
            needs_layout_passes=False, use_tc_tiling_on_sc=False),
        scratch_types=[
            pltpu.VMEM((W,), jnp.float32),
            pltpu.VMEM((W,), jnp.float32),
            pltpu.VMEM((HB, 16), jnp.int32),
            pltpu.VMEM((NCHUNK, 16), jnp.float32),
            pltpu.VMEM((HB,), jnp.int32),
            pltpu.VMEM((HB,), jnp.int32),
            pltpu.VMEM((CAP1,), jnp.float32),
            pltpu.VMEM((CAP1,), jnp.int32),
            pltpu.VMEM((CAPF,), jnp.float32),
            pltpu.VMEM((CAPF,), jnp.int32),
            pltpu.VMEM((CAPF // 128, 128), jnp.int32),
            pltpu.VMEM((CAPF, 16), jnp.float32),
            pltpu.VMEM((4, CAPF), jnp.float32),
            pltpu.SemaphoreType.DMA,
            pltpu.SemaphoreType.DMA,
            pltpu.SemaphoreType.DMA,
        ],
    )(logits_flat, boxes_tbl)


def _tc_body(score_ref, idx_ref, boxes_ref, size_ref,
             lab_ref, box_ref, sc_ref):
    s = score_ref[0, 0]
    idx = idx_ref[0, 0]
    idxf = idx.astype(jnp.float32)

    srow = s[None, :]
    scol = s[:, None]
    irow = idxf[None, :]
    icol = idxf[:, None]
    better = jnp.logical_or(
        srow > scol, jnp.logical_and(srow == scol, irow < icol))
    ranks = jnp.sum(better.astype(jnp.int32), axis=1)

    prow = lax.broadcasted_iota(jnp.int32, (CAPF, CAPF), 0)
    onehot = (prow == ranks[None, :]).astype(jnp.float32)

    def pick(col):
        return jnp.sum(onehot * col[None, :], axis=1)

    bx = boxes_ref[0]
    top_s = pick(s)[:NQ]
    top_i = pick(idxf)[:NQ].astype(jnp.int32)
    labels = top_i % C

    cx = pick(bx[:, 0])[:NQ]
    cy = pick(bx[:, 1])[:NQ]
    w = pick(bx[:, 2])[:NQ]
    h = pick(bx[:, 3])[:NQ]
    xyxy = jnp.stack(
        [cx - 0.5 * w, cy - 0.5 * h, cx + 0.5 * w, cy + 0.5 * h], axis=-1)
    w0 = size_ref[0, 0, 0]
    h0 = size_ref[0, 0, 1]
    scale = jnp.stack([w0, h0, w0, h0])[None, :]
    xyxy = xyxy * scale

    lab_ref[0, 0] = labels
    box_ref[0] = xyxy
    sc_ref[0, 0] = top_s


@jax.jit
def _tc_finalize(scores, idxs, boxes, sizes):
    labels, xyxy, top_s = pl.pallas_call(
        _tc_body,
        grid=(B,),
        in_specs=[
            pl.BlockSpec((1, 1, CAPF), lambda r: (r, 0, 0)),
            pl.BlockSpec((1, 1, CAPF), lambda r: (r, 0, 0)),
            pl.BlockSpec((1, CAPF, 4), lambda r: (r, 0, 0)),
            pl.BlockSpec((1, 1, 2), lambda r: (r, 0, 0)),
        ],
        out_specs=[
            pl.BlockSpec((1, 1, NQ), lambda r: (r, 0, 0)),
            pl.BlockSpec((1, NQ, 4), lambda r: (r, 0, 0)),
            pl.BlockSpec((1, 1, NQ), lambda r: (r, 0, 0)),
        ],
        out_shape=[
            jax.ShapeDtypeStruct((B, 1, NQ), jnp.int32),
            jax.ShapeDtypeStruct((B, NQ, 4), jnp.float32),
            jax.ShapeDtypeStruct((B, 1, NQ), jnp.float32),
        ],
    )(scores.reshape(B, 1, CAPF), idxs.reshape(B, 1, CAPF), boxes,
      sizes.reshape(B, 1, 2))
    return labels.reshape(B, NQ), xyxy, top_s.reshape(B, NQ)


def kernel(pred_logits, pred_boxes, orig_target_sizes):
    logits_flat = pred_logits.reshape(B * ROW)
    boxes_tbl = pred_boxes.reshape(B * N // 4, 16)
    cand_v, cand_i, cand_b = _sc_select(logits_flat, boxes_tbl)
    cand_b = cand_b.reshape(B, 4, CAPF).transpose(0, 2, 1)
    cand_s = jax.nn.sigmoid(cand_v)
    labels, boxes, scores = _tc_finalize(cand_s, cand_i, cand_b,
                                         orig_target_sizes)
    return (labels, boxes, scores)
